# Initial kernel scaffold; baseline (speedup 1.0000x reference)
import jax
import jax.numpy as jnp
from jax.experimental import pallas as pl


def kernel(coords, adj_0, t, epsilon, node_w, node_b, edge_w, edge_b, tw1, tb1, tw2, tb2, U, V, W, A, Bm, C, Tp, out_w, out_b):
    raise NotImplementedError("write your pallas kernel here")


# speedup vs baseline: 1.1387x; 1.1387x over previous
import numpy as np
import jax
import jax.numpy as jnp
from jax.experimental import pallas as pl
from jax.experimental.pallas import tpu as pltpu

_H = 64
_L = 6
_B = 2
_N = 200
_TI = 40
_CPB = _N // _TI
_CH = _TI * _N

_f32 = jnp.float32
_bf16 = jnp.bfloat16


def _mm(a, w):
    return jax.lax.dot_general(
        a.astype(_bf16), w.astype(_bf16),
        dimension_numbers=(((1,), (0,)), ((), ())),
        preferred_element_type=_f32)


def _ln(v):
    m = jnp.mean(v, axis=-1, keepdims=True)
    s = jnp.mean((v - m) ** 2, axis=-1, keepdims=True)
    return (v - m) * jax.lax.rsqrt(s + 1e-5)


def _body(adj_ref, eps_ref, coords_ref, t_ref, nw_ref, nb_ref, ew_ref, eb_ref,
          tw1_ref, tb1_ref, tw2_ref, tb2_ref, U_ref, V_ref, W_ref, A_ref,
          Bm_ref, C_ref, Tp_ref, ow_ref, ob_ref, out_ref,
          e_s, x_s, ag_s, vx_s, wx_s, cx_s, temb_s, tadd_s):
    c0 = coords_ref[:, 0:1]
    c1 = coords_ref[:, 1:2]
    x_s[...] = c0 * nw_ref[0:1, :] + c1 * nw_ref[1:2, :] + nb_ref[...]

    half = _H // 2
    j = jax.lax.broadcasted_iota(jnp.int32, (1, half), 1).astype(_f32)
    freqs = jnp.exp(-(np.log(10000.0) / half) * j)
    for b in range(_B):
        args = t_ref[b] * 1000.0 * freqs
        emb = jnp.concatenate([jnp.sin(args), jnp.cos(args)], axis=1)
        h1 = jnp.maximum(_mm(emb, tw1_ref[...]) + tb1_ref[...], 0.0)
        temb_s[pl.ds(b, 1), :] = _mm(h1, tw2_ref[...]) + tb2_ref[...]

    ew3 = ew_ref[...].reshape(1, 1, _H)
    eb3 = eb_ref[...].reshape(1, 1, _H)

    for b in range(_B):
        tt = t_ref[b]

        def init_c(c, carry, b=b, tt=tt):
            a = adj_ref[pl.ds(b * _N + c * _TI, _TI), :]
            ep = eps_ref[pl.ds(b * _N + c * _TI, _TI), :]
            adjt = (1.0 - tt) * (a * 2.0 - 1.0) + tt * ep
            e0 = adjt[:, :, None] * ew3 + eb3
            e_s[pl.ds(b * _N * _N + c * _CH, _CH), :] = e0.reshape(_CH, _H)
            return carry

        jax.lax.fori_loop(0, _CPB, init_c, 0)

    for l in range(_L):
        xv = x_s[...]
        vx_s[...] = _mm(xv, V_ref[l])
        wx_s[...] = _mm(xv, W_ref[l])
        cx_s[...] = _mm(xv, C_ref[l])
        tadd_s[...] = _mm(temb_s[...], Tp_ref[l])
        Ul = U_ref[l].astype(_bf16)

        for b in range(_B):
            wx = wx_s[b * _N:(b + 1) * _N, :]
            cx = cx_s[b * _N:(b + 1) * _N, :]
            td = tadd_s[b:b + 1, :].reshape(1, 1, _H)

            def chunk(c, carry, b=b, wx=wx, cx=cx, td=td):
                ech = e_s[pl.ds(b * _N * _N + c * _CH, _CH), :]
                eU = jax.lax.dot_general(
                    ech.astype(_bf16), Ul,
                    dimension_numbers=(((1,), (0,)), ((), ())),
                    preferred_element_type=_f32)
                vx = vx_s[pl.ds(b * _N + c * _TI, _TI), :]
                en = (eU.reshape(_TI, _N, _H) + vx[:, None, :]
                      + wx[None, :, :] + td)
                g = 1.0 / (1.0 + jnp.exp(-en))
                num = jnp.sum(g * cx[None, :, :], axis=1)
                den = jnp.sum(g, axis=1)
                ag_s[pl.ds(b * _N + c * _TI, _TI), :] = num / (den + 1e-6)
                e_s[pl.ds(b * _N * _N + c * _CH, _CH), :] = (
                    ech + jnp.maximum(_ln(en), 0.0).reshape(_CH, _H))
                return carry

            jax.lax.fori_loop(0, _CPB, chunk, 0)
        xa = _mm(x_s[...], A_ref[l]) + _mm(ag_s[...], Bm_ref[l])
        x_s[...] = x_s[...] + jnp.maximum(_ln(xa), 0.0)

    ow3 = ow_ref[...].reshape(1, 1, _H)
    ob = ob_ref[0]

    loss = _f32(0.0)
    for b in range(_B):

        def loss_c(c, acc, b=b):
            e3 = e_s[pl.ds(b * _N * _N + c * _CH, _CH), :].reshape(
                _TI, _N, _H)
            pv = jnp.sum(e3 * ow3, axis=-1) + ob
            a = adj_ref[pl.ds(b * _N + c * _TI, _TI), :]
            ep = eps_ref[pl.ds(b * _N + c * _TI, _TI), :]
            d = pv - (ep - (a * 2.0 - 1.0))
            return acc + jnp.sum(d * d)

        loss = jax.lax.fori_loop(0, _CPB, loss_c, loss)
    out_ref[...] = (loss * (1.0 / (_B * _N * _N))).reshape(1, 1)


def kernel(coords, adj_0, t, epsilon, node_w, node_b, edge_w, edge_b, tw1,
           tb1, tw2, tb2, U, V, W, A, Bm, C, Tp, out_w, out_b):
    vmem = pl.BlockSpec(memory_space=pltpu.VMEM)
    smem = pl.BlockSpec(memory_space=pltpu.SMEM)
    out = pl.pallas_call(
        _body,
        out_shape=jax.ShapeDtypeStruct((1, 1), _f32),
        in_specs=[vmem, vmem, vmem, smem, vmem, vmem, vmem, vmem, vmem,
                  vmem, vmem, vmem, vmem, vmem, vmem, vmem, vmem, vmem,
                  vmem, vmem, smem],
        out_specs=vmem,
        scratch_shapes=[
            pltpu.VMEM((_B * _N * _N, _H), _f32),
            pltpu.VMEM((_B * _N, _H), _f32),
            pltpu.VMEM((_B * _N, _H), _f32),
            pltpu.VMEM((_B * _N, _H), _f32),
            pltpu.VMEM((_B * _N, _H), _f32),
            pltpu.VMEM((_B * _N, _H), _f32),
            pltpu.VMEM((_B, _H), _f32),
            pltpu.VMEM((_B, _H), _f32),
        ],
    )(adj_0.reshape(_B * _N, _N), epsilon.reshape(_B * _N, _N),
      coords.reshape(_B * _N, 2), t, node_w, node_b.reshape(1, _H),
      edge_w, edge_b.reshape(1, _H), tw1, tb1.reshape(1, _H), tw2,
      tb2.reshape(1, _H), U, V, W, A, Bm, C, Tp, out_w.reshape(1, _H),
      out_b)
    return out.reshape(())
